# Initial kernel scaffold; baseline (speedup 1.0000x reference)
import jax
import jax.numpy as jnp
from jax.experimental import pallas as pl


def kernel(atom_types, system_ids, weights):
    raise NotImplementedError("write your pallas kernel here")


# speedup vs baseline: 317.3343x; 317.3343x over previous
import functools

import jax
import jax.numpy as jnp
from jax import lax
from jax.experimental import pallas as pl
from jax.experimental.pallas import tpu as pltpu
from jax.experimental.pallas import tpu_sc as plsc

N_AT = 4_000_000
N_TY = 100
N_SY = 16384

NC = 2
NS = 16
NW = NC * NS
LANES = 16

PER_TILE = N_AT // NW
N_CHUNK = 6
CH = 20832
VR = CH // LANES
TAIL = PER_TILE - N_CHUNK * CH


def _sc_partials(atom_types, system_ids, weights):
    mesh = plsc.VectorSubcoreMesh(core_axis_name="c", subcore_axis_name="s")

    @functools.partial(
        pl.kernel,
        out_type=jax.ShapeDtypeStruct((NW, N_SY), jnp.float32),
        mesh=mesh,
        compiler_params=pltpu.CompilerParams(needs_layout_passes=False),
        scratch_types=[
            pltpu.VMEM((N_SY,), jnp.float32),
            pltpu.VMEM((CH,), jnp.int32),
            pltpu.VMEM((CH,), jnp.int32),
            pltpu.VMEM((CH,), jnp.int32),
            pltpu.VMEM((CH,), jnp.int32),
            pltpu.VMEM((N_TY,), jnp.float32),
            pltpu.VMEM((16,), jnp.int32),
            pltpu.VMEM((16,), jnp.int32),
            pltpu.SemaphoreType.DMA,
            pltpu.SemaphoreType.DMA,
        ],
    )
    def k(at_hbm, sid_hbm, w_hbm, out_hbm,
          acc, atb0, atb1, sidb0, sidb1, wtab, att, sidt, sem0, sem1):
        c = lax.axis_index("c")
        s = lax.axis_index("s")
        wid = c * NS + s
        base = wid * PER_TILE
        sems = (sem0, sem1)
        atbs = (atb0, atb1)
        sidbs = (sidb0, sidb1)

        def start(ci, slot):
            off = base + ci * CH
            pltpu.async_copy(at_hbm.at[pl.ds(off, CH)], atbs[slot], sems[slot])
            pltpu.async_copy(sid_hbm.at[pl.ds(off, CH)], sidbs[slot], sems[slot])

        def wait(ci, slot):
            off = base + ci * CH
            pltpu.make_async_copy(at_hbm.at[pl.ds(off, CH)], atbs[slot], sems[slot]).wait()
            pltpu.make_async_copy(sid_hbm.at[pl.ds(off, CH)], sidbs[slot], sems[slot]).wait()

        start(0, 0)
        start(1, 1)
        pltpu.sync_copy(w_hbm.at[0], wtab)

        zero16 = jnp.zeros((LANES,), jnp.float32)

        def zbody(i, carry):
            acc[pl.ds(i * LANES, LANES)] = zero16
            return carry

        lax.fori_loop(0, N_SY // LANES, zbody, 0, unroll=8)

        lane = lax.iota(jnp.int32, LANES)
        lvec = lane * VR

        def process(slot):
            atc = atbs[slot]
            sidc = sidbs[slot]

            def ibody(i, carry):
                idx = lvec + i
                av = plsc.load_gather(atc, [idx])
                sv = plsc.load_gather(sidc, [idx])
                wv = plsc.load_gather(wtab, [av])
                plsc.addupdate_scatter(acc, [sv], wv)
                return carry

            lax.fori_loop(0, VR, ibody, 0, unroll=8)

        for ci in range(N_CHUNK):
            slot = ci % 2
            wait(ci, slot)
            process(slot)
            if ci + 2 < N_CHUNK:
                start(ci + 2, slot)

        toff = base + PER_TILE - 16
        pltpu.sync_copy(at_hbm.at[pl.ds(toff, 16)], att)
        pltpu.sync_copy(sid_hbm.at[pl.ds(toff, 16)], sidt)
        wvt = plsc.load_gather(wtab, [att[...]])
        plsc.addupdate_scatter(acc, [sidt[...]], wvt, mask=lane >= (16 - TAIL))

        pltpu.sync_copy(acc, out_hbm.at[wid])

    return k(atom_types, system_ids, weights)


def _tc_reduce(partials):
    def body(p_ref, o_ref):
        o_ref[...] = jnp.sum(p_ref[...], axis=0, keepdims=True)

    cols = 2048
    return pl.pallas_call(
        body,
        grid=(N_SY // cols,),
        in_specs=[pl.BlockSpec((NW, cols), lambda i: (0, i))],
        out_specs=pl.BlockSpec((1, cols), lambda i: (0, i)),
        out_shape=jax.ShapeDtypeStruct((1, N_SY), jnp.float32),
    )(partials)


@jax.jit
def kernel(atom_types, system_ids, weights):
    partials = _sc_partials(atom_types, system_ids, weights)
    per_system = _tc_reduce(partials)
    return per_system.reshape(-1, 1)
